# baseline (speedup 1.0000x reference)
import functools

import jax
import jax.numpy as jnp
from jax import lax
from jax.experimental import pallas as pl
from jax.experimental.pallas import tpu as pltpu
from jax.experimental.pallas import tpu_sc as plsc

NUM_NTYPE = 4
EMBED = 32
N = 16384
VOCAB = 1000000

_info = plsc.get_sparse_core_info()
NC, NS, L = _info.num_cores, _info.num_subcores, _info.num_lanes
NW = NC * NS
B_PER_W = N // NW
VREGS = B_PER_W // L
CHUNK = 128
NCHUNK = B_PER_W // CHUNK + NUM_NTYPE
PAD = NCHUNK * CHUNK


def _embed_kernel(node_tids, type_ids, e0, e1, e2, e3, out_t,
                  tid_v, typ_v, cidx_v, islot_v, plane_v, outloc_v, gsem):
    tables = (e0, e1, e2, e3)
    wid = lax.axis_index("s") * NC + lax.axis_index("c")
    base = wid * B_PER_W

    pltpu.sync_copy(node_tids.at[pl.ds(base, B_PER_W)], tid_v)
    pltpu.sync_copy(type_ids.at[pl.ds(base, B_PER_W)], typ_v)

    lane = lax.iota(jnp.int32, L)
    for c in range(PAD // L):
        cidx_v[pl.ds(c * L, L)] = lane * 61 + (c * 977 + wid * 31013)

    off = jnp.int32(0)
    bounds = [jnp.int32(0)]
    for t in range(NUM_NTYPE):
        cnt = jnp.int32(0)
        for v in range(VREGS):
            tid = tid_v[pl.ds(v * L, L)]
            typ = typ_v[pl.ds(v * L, L)]
            m = tid == t
            mi = m.astype(jnp.int32)
            slot = (off + cnt - 1) + plsc.cumsum(mi)
            plsc.store_scatter(cidx_v, [slot], typ, mask=m)
            plsc.store_scatter(islot_v, [lane + v * L], slot, mask=m)
            cnt = cnt + jnp.sum(mi)
        off = off + ((cnt + (CHUNK - 1)) >> 7 << 7)
        bounds.append(off >> 7)

    for t in range(NUM_NTYPE):
        def gbody(c, carry, t=t):
            idxs = cidx_v.at[pl.ds(c * CHUNK, CHUNK)]
            for f in range(EMBED):
                pltpu.async_copy(
                    tables[t].at[f].at[idxs],
                    plane_v.at[f, pl.ds(c * CHUNK, CHUNK)], gsem)
            return carry
        lax.fori_loop(bounds[t], bounds[t + 1], gbody, jnp.int32(0))
    for t in range(NUM_NTYPE):
        def wbody(c, carry, t=t):
            idxs = cidx_v.at[pl.ds(c * CHUNK, CHUNK)]
            for f in range(EMBED):
                pltpu.make_async_copy(
                    tables[t].at[f].at[idxs],
                    plane_v.at[f, pl.ds(c * CHUNK, CHUNK)], gsem).wait()
            return carry
        lax.fori_loop(bounds[t], bounds[t + 1], wbody, jnp.int32(0))

    for v in range(VREGS):
        isl = islot_v[pl.ds(v * L, L)]
        for f in range(EMBED):
            vals = plsc.load_gather(plane_v.at[f], [isl])
            outloc_v[f, pl.ds(v * L, L)] = vals

    pltpu.sync_copy(outloc_v, out_t.at[:, pl.ds(base, B_PER_W)])


@jax.jit
def _run(node_tids, type_ids, emb0, emb1, emb2, emb3):
    mesh = plsc.VectorSubcoreMesh(core_axis_name="c", subcore_axis_name="s")
    f = functools.partial(
        pl.kernel,
        mesh=mesh,
        compiler_params=pltpu.CompilerParams(
            needs_layout_passes=False, use_tc_tiling_on_sc=False),
        out_type=jax.ShapeDtypeStruct((EMBED, N), jnp.float32),
        scratch_types=[
            pltpu.VMEM((B_PER_W,), jnp.int32),
            pltpu.VMEM((B_PER_W,), jnp.int32),
            pltpu.VMEM((PAD,), jnp.int32),
            pltpu.VMEM((B_PER_W,), jnp.int32),
            pltpu.VMEM((EMBED, PAD), jnp.float32),
            pltpu.VMEM((EMBED, B_PER_W), jnp.float32),
            pltpu.SemaphoreType.DMA,
        ],
    )(_embed_kernel)
    out_t = f(node_tids, type_ids,
              jnp.swapaxes(emb0, 0, 1), jnp.swapaxes(emb1, 0, 1),
              jnp.swapaxes(emb2, 0, 1), jnp.swapaxes(emb3, 0, 1))
    return jnp.swapaxes(out_t, 0, 1)


def kernel(node_ids, node_tids, type_ids, emb0, emb1, emb2, emb3):
    del node_ids
    return _run(node_tids.astype(jnp.int32), type_ids.astype(jnp.int32),
                emb0, emb1, emb2, emb3)
